# baseline (speedup 1.0000x reference)
import jax
import jax.numpy as jnp
from jax import lax
from jax.experimental import pallas as pl
from jax.experimental.pallas import tpu as pltpu
from jax.experimental.pallas import tpu_sc as plsc

_DIM = 32
_SEQ = 50
_PAD = 128
_NB = 16
_NW = 32
_TOK = _NB * _SEQ


def _gather_body(table_hbm, idx_hbm, out_hbm, idx_v, idx_c, rows_v, gsem):
    n_batch = out_hbm.shape[0]
    bat_w = n_batch // _NW
    n_chunk = bat_w // _NB
    wid = lax.axis_index("s") * 2 + lax.axis_index("c")
    b_w = wid * bat_w
    lane = lax.iota(jnp.int32, 16)

    def body(g, carry):
        b0 = pl.multiple_of(b_w + g * _NB, _NB)
        pltpu.sync_copy(idx_hbm.at[pl.ds(b0 * _PAD, _NB * _PAD)], idx_v)
        for t in range(0, _NB * 56, 16):
            h = t + lane
            j = (h * 1171) >> 16
            tok = h - j * 6
            row = (tok * 1311) >> 16
            src = jnp.minimum(tok + row * (_PAD - _SEQ), _NB * _PAD - 1)
            idx_c[pl.ds(t, 16)] = plsc.load_gather(idx_v, [src])
        copies = []
        for j in range(_NB):
            copies.append(
                pltpu.async_copy(
                    table_hbm.at[idx_c.at[pl.ds(j * 56, _SEQ)]],
                    rows_v.at[j],
                    gsem,
                )
            )
        for c in copies:
            c.wait()
        pltpu.sync_copy(rows_v, out_hbm.at[pl.ds(b0, _NB)])
        return carry

    lax.fori_loop(0, n_chunk, body, 0)


import functools


@functools.partial(jax.jit, static_argnums=2)
def _sc_gather(idx_flat, weight, n_batch):
    mesh = plsc.VectorSubcoreMesh(core_axis_name="c", subcore_axis_name="s")
    return pl.kernel(
        _gather_body,
        out_type=jax.ShapeDtypeStruct((n_batch, _SEQ, _DIM), jnp.float32),
        mesh=mesh,
        scratch_types=[
            pltpu.VMEM((_NB * _PAD,), jnp.int32),
            pltpu.VMEM((_NB * 56,), jnp.int32),
            pltpu.VMEM((_NB, _SEQ, _DIM), jnp.float32),
            pltpu.SemaphoreType.DMA,
        ],
        compiler_params=pltpu.CompilerParams(use_tc_tiling_on_sc=False, needs_layout_passes=False),
    )(weight, idx_flat)


def kernel(indices, weight):
    n_batch = indices.shape[0]
    idx_pad = jnp.pad(indices.astype(jnp.int32), ((0, 0), (0, _PAD - _SEQ)))
    return _sc_gather(idx_pad.reshape(-1), weight, n_batch)
